# baseline (speedup 1.0000x reference)
import functools

import jax
import jax.numpy as jnp
from jax import lax
from jax.experimental import pallas as pl
from jax.experimental.pallas import tpu as pltpu
from jax.experimental.pallas import tpu_sc as plsc

_NUM_FIELDS = 26
_VOCAB = 100000
_EMBED_DIM = 32
_NUM_NUMERICAL = 13
_CHUNK = 32
_LINES_PER_VOCAB = _VOCAB // 4


def _build(batch):
    info = plsc.get_sparse_core_info()
    n_workers = info.num_cores * info.num_subcores
    b_per_w = batch // n_workers
    n_chunks = b_per_w // _CHUNK
    emb_d = _NUM_FIELDS * _EMBED_DIM
    mesh = plsc.VectorSubcoreMesh(core_axis_name="c", subcore_axis_name="s")

    @functools.partial(
        pl.kernel,
        mesh=mesh,
        out_type=jax.ShapeDtypeStruct((batch, emb_d), jnp.float32),
        compiler_params=pltpu.CompilerParams(needs_layout_passes=False),
        scratch_types=[
            pltpu.VMEM((_NUM_FIELDS, b_per_w), jnp.int32),
            pltpu.VMEM((_CHUNK,), jnp.int32),
            pltpu.VMEM((_CHUNK,), jnp.int32),
            pltpu.VMEM((_CHUNK, 128), jnp.float32),
            pltpu.VMEM((_CHUNK, 128), jnp.float32),
            pltpu.VMEM((_CHUNK, emb_d), jnp.float32),
            pltpu.VMEM((_CHUNK, emb_d), jnp.float32),
            pltpu.SemaphoreType.DMA,
            pltpu.SemaphoreType.DMA,
            pltpu.SemaphoreType.DMA,
            pltpu.SemaphoreType.DMA,
        ],
    )
    def flattener(idx_hbm, tab_hbm, out_hbm, raw_v, jbuf0, jbuf1,
                  stage0, stage1, asm0, asm1, gsem0, gsem1, wsem0, wsem1):
        jbufs = (jbuf0, jbuf1)
        stages = (stage0, stage1)
        asms = (asm0, asm1)
        gsems = (gsem0, gsem1)
        wsems = (wsem0, wsem1)
        wid = lax.axis_index("s") * info.num_cores + lax.axis_index("c")
        base = wid * b_per_w

        pltpu.sync_copy(
            idx_hbm.at[:, pl.ds(pl.multiple_of(base, b_per_w), b_per_w)],
            raw_v)

        iota = lax.iota(jnp.int32, 16)

        def raw_slice(f, c, g):
            off = pl.multiple_of(c * _CHUNK, _CHUNK) + 16 * g
            return raw_v[f, pl.ds(off, 16)]

        def fill_jbuf(f, c, fh):
            line_base = f * _LINES_PER_VOCAB
            for g in range(_CHUNK // 16):
                codes = raw_slice(f, c, g)
                jbufs[fh][pl.ds(16 * g, 16)] = (
                    lax.shift_right_logical(codes, 2) + line_base)

        def gather_start(f, c, fh):
            fill_jbuf(f, c, fh)
            return pltpu.async_copy(
                tab_hbm.at[jbufs[fh]], stages[fh], gsems[fh])

        def gather_wait(fh):
            pltpu.make_async_copy(
                tab_hbm.at[pl.ds(0, _CHUNK), :], stages[fh],
                gsems[fh]).wait()

        def fixup(f, c, fh, h):
            stage, asm = stages[fh], asms[h]
            for g in range(_CHUNK // 16):
                rows = iota + 16 * g
                s_off = lax.shift_left(
                    lax.bitwise_and(raw_slice(f, c, g), 3), 5)
                col0 = f * _EMBED_DIM
                for j in range(_EMBED_DIM):
                    vals = plsc.load_gather(stage, [rows, s_off + j])
                    plsc.store_scatter(
                        asm, [rows, jnp.zeros((16,), jnp.int32) + (col0 + j)],
                        vals)

        def chunk_body(c, h):
            gather_start(0, c, 0)
            gather_start(1, c, 1)

            @pl.loop(0, _NUM_FIELDS, step=2)
            def _(g):
                for fh in range(2):
                    f = g + fh
                    gather_wait(fh)
                    fixup(f, c, fh, h)

                    @pl.when(f + 2 < _NUM_FIELDS)
                    def _():
                        gather_start(f + 2, c, fh)

            row = base + c * _CHUNK
            return pltpu.async_copy(
                asms[h], out_hbm.at[pl.ds(row, _CHUNK), :], wsems[h])

        def write_wait(h):
            pltpu.make_async_copy(
                asms[h], out_hbm.at[pl.ds(0, _CHUNK), :], wsems[h]).wait()

        @pl.loop(0, n_chunks, step=2)
        def _(c):
            for h in range(2):
                @pl.when(c + h >= 2)
                def _():
                    write_wait(h)
                chunk_body(c + h, h)

        for h in range(2):
            write_wait(h)

    return flattener


def kernel(numerical, cat_indices, tables):
    batch = numerical.shape[0]
    tab_lines = tables.reshape(_NUM_FIELDS * _LINES_PER_VOCAB, 128)
    idx_t = cat_indices.astype(jnp.int32).T
    emb = _build(batch)(idx_t, tab_lines)
    return jnp.concatenate([numerical, emb], axis=1)
